# baseline (speedup 1.0000x reference)
import functools

import jax
import jax.numpy as jnp
from jax import lax
from jax.experimental import pallas as pl
from jax.experimental.pallas import tpu as pltpu
from jax.experimental.pallas import tpu_sc as plsc

NUM_EMBEDDINGS = 1000000
EMBED_DIM = 32
NUM_FEATURES = 2
BATCH = 4096
MAX_LEN = 20

LANES = 16
NUM_WORKERS = 32
BAGS_PER_WORKER = BATCH // NUM_WORKERS
IDX_PER_WORKER = BAGS_PER_WORKER * MAX_LEN
GATHER_BATCH = 128
GATHERS_PER_FEATURE = IDX_PER_WORKER // GATHER_BATCH


def _sc_body(idx_hbm, t0_hbm, t1_hbm, wv_hbm, out_hbm,
             idx_v, rows_v, acc_v, wv_v, sem):
    cid = lax.axis_index("c")
    sid = lax.axis_index("s")
    wid = sid * 2 + cid
    base_bag = wid * BAGS_PER_WORKER

    pltpu.sync_copy(wv_hbm, wv_v)

    for f in range(NUM_FEATURES):
        table = t0_hbm if f == 0 else t1_hbm

        pltpu.sync_copy(
            idx_hbm.at[pl.ds(f * BATCH * MAX_LEN + wid * IDX_PER_WORKER,
                             IDX_PER_WORKER)],
            idx_v)

        copies = []
        for j in range(GATHERS_PER_FEATURE):
            copies.append(
                pltpu.async_copy(
                    table.at[idx_v.at[pl.ds(j * GATHER_BATCH, GATHER_BATCH)]],
                    rows_v.at[pl.ds(j * GATHER_BATCH, GATHER_BATCH)],
                    sem))
        for c in copies:
            c.wait()

        w = tuple(wv_v[pl.ds((f * MAX_LEN + l) * LANES, LANES)]
                  for l in range(MAX_LEN))

        def bag_body(b, w):
            r0 = b * MAX_LEN
            a0e = w[0] * rows_v[r0, pl.ds(0, LANES)]
            a1e = w[0] * rows_v[r0, pl.ds(LANES, LANES)]
            a0o = w[1] * rows_v[r0 + 1, pl.ds(0, LANES)]
            a1o = w[1] * rows_v[r0 + 1, pl.ds(LANES, LANES)]
            for l in range(2, MAX_LEN, 2):
                a0e = a0e + w[l] * rows_v[r0 + l, pl.ds(0, LANES)]
                a1e = a1e + w[l] * rows_v[r0 + l, pl.ds(LANES, LANES)]
                a0o = a0o + w[l + 1] * rows_v[r0 + l + 1, pl.ds(0, LANES)]
                a1o = a1o + w[l + 1] * rows_v[r0 + l + 1, pl.ds(LANES, LANES)]
            o = pl.multiple_of(b * EMBED_DIM, EMBED_DIM)
            acc_v[pl.ds(o, LANES)] = a0e + a0o
            acc_v[pl.ds(o + LANES, LANES)] = a1e + a1o
            return w

        lax.fori_loop(0, BAGS_PER_WORKER, bag_body, w, unroll=False)

        pltpu.sync_copy(
            acc_v,
            out_hbm.at[pl.ds(f * BATCH * EMBED_DIM + base_bag * EMBED_DIM,
                             BAGS_PER_WORKER * EMBED_DIM)])


@jax.jit
def _fpebc(idx_flat, table0, table1, wv):
    mesh = plsc.VectorSubcoreMesh(core_axis_name="c", subcore_axis_name="s")
    kern = functools.partial(
        pl.kernel,
        out_type=jax.ShapeDtypeStruct((NUM_FEATURES * BATCH * EMBED_DIM,),
                                      jnp.float32),
        mesh=mesh,
        compiler_params=pltpu.CompilerParams(use_tc_tiling_on_sc=False),
        scratch_types=[
            pltpu.VMEM((IDX_PER_WORKER,), jnp.int32),
            pltpu.VMEM((IDX_PER_WORKER, EMBED_DIM), jnp.float32),
            pltpu.VMEM((BAGS_PER_WORKER * EMBED_DIM,), jnp.float32),
            pltpu.VMEM((NUM_FEATURES * MAX_LEN * LANES,), jnp.float32),
            pltpu.SemaphoreType.DMA,
        ],
    )(_sc_body)
    out_flat = kern(idx_flat, table0, table1, wv)
    return (out_flat.reshape(NUM_FEATURES, BATCH, EMBED_DIM)
            .transpose(1, 0, 2)
            .reshape(BATCH, NUM_FEATURES * EMBED_DIM))


def kernel(indices, table0, table1, pos_w):
    idx_flat = indices.reshape(-1)
    wv = jnp.broadcast_to(pos_w[:, :, None],
                          (NUM_FEATURES, MAX_LEN, LANES)).reshape(-1)
    return _fpebc(idx_flat, table0, table1, wv)
